# Initial kernel scaffold; baseline (speedup 1.0000x reference)
import jax
import jax.numpy as jnp
from jax.experimental import pallas as pl


def kernel(p, f, sample_idx, c1w1, g1, b1, c1w2, g2, b2, c2w1, g3, b3, c2w2, g4, b4):
    raise NotImplementedError("write your pallas kernel here")


# speedup vs baseline: 8.9078x; 8.9078x over previous
import functools

import jax
import jax.numpy as jnp
import numpy as np
from jax import lax
from jax.experimental import pallas as pl
from jax.experimental.pallas import tpu as pltpu
from jax.experimental.pallas import tpu_sc as plsc

B = 2
N = 8192
M = N // 4
NS = 32
CIN = 128
CMID = 128
COUT = 256
RADIUS = 0.1
EPS = 1e-5
BN_ = B * N
NT2 = B * M * NS
PAD = 128

R2 = np.float32(RADIUS * RADIUS)

TN1 = 2048
TN2 = 1024
TM = 128
RT = 1024

NC = 2
NSUB = 16
NW = NC * NSUB
CH = 128


def _k_stats1(w1_ref, f_ref, sq_ref):
    z = jnp.dot(w1_ref[...], f_ref[...], preferred_element_type=jnp.float32)
    @pl.when(pl.program_id(0) == 0)
    def _():
        sq_ref[...] = jnp.zeros_like(sq_ref)
    sq_ref[0:1, :] += jnp.sum(z, axis=1, keepdims=True).T
    sq_ref[1:2, :] += jnp.sum(z * z, axis=1, keepdims=True).T


def _k_mlp1(w1a_ref, c1_ref, w2_ref, f_ref, z2t_ref, sq_ref):
    z1 = jnp.dot(w1a_ref[...], f_ref[...], preferred_element_type=jnp.float32)
    r = jnp.maximum(z1 + c1_ref[...], 0.0)
    z2t = lax.dot_general(r, w2_ref[...], (((0,), (1,)), ((), ())),
                          preferred_element_type=jnp.float32)
    z2t_ref[...] = z2t
    @pl.when(pl.program_id(0) == 0)
    def _():
        sq_ref[...] = jnp.zeros_like(sq_ref)
    sq_ref[0:1, :] += jnp.sum(z2t, axis=0, keepdims=True)
    sq_ref[1:2, :] += jnp.sum(z2t * z2t, axis=0, keepdims=True)


def _k_ball(pt_ref, c_ref, idx_ref):
    b = pl.program_id(0)
    pt = pt_ref[0]
    d = None
    for c in range(3):
        diff = c_ref[:, c:c + 1] - pt[c:c + 1, :]
        sq = diff * diff
        d = sq if d is None else d + sq
    hits = (d <= R2).astype(jnp.int32)
    cs = hits
    sh = 1
    while sh < N:
        cs = cs + jnp.concatenate(
            [jnp.zeros((TM, sh), jnp.int32), cs[:, :N - sh]], axis=1)
        sh *= 2
    iota = lax.broadcasted_iota(jnp.int32, (TM, N), 1)
    cols = []
    for k in range(NS):
        cand = jnp.where(cs >= (k + 1), iota, N)
        cols.append(jnp.min(cand, axis=1, keepdims=True))
    idx = jnp.concatenate(cols, axis=1)
    first = idx[:, 0:1]
    idx = jnp.where(idx == N, first, idx)
    idx_ref[...] = idx + b * N


def _dp_tile(gx_ref, np_ref):
    gx = gx_ref[...]
    npv = np_ref[...]
    dp = (gx.reshape(RT // NS, NS, PAD) - npv[:, None, :]) / jnp.float32(RADIUS)
    return dp.reshape(RT, PAD)


def _k_dpstats(gx_ref, np_ref, s_ref, c_ref):
    dp = _dp_tile(gx_ref, np_ref)
    @pl.when(pl.program_id(0) == 0)
    def _():
        s_ref[...] = jnp.zeros_like(s_ref)
        c_ref[...] = jnp.zeros_like(c_ref)
    s_ref[0:1, :] += jnp.sum(dp, axis=0, keepdims=True)
    c_ref[...] += lax.dot_general(dp, dp, (((0,), (0,)), ((), ())),
                                  preferred_element_type=jnp.float32)


def _k_r3stats(gx_ref, np_ref, w3_ref, c3_ref, s_ref, c_ref):
    dp = _dp_tile(gx_ref, np_ref)
    r3 = jnp.maximum(
        lax.dot_general(dp, w3_ref[...], (((1,), (1,)), ((), ())),
                        preferred_element_type=jnp.float32) + c3_ref[...], 0.0)
    @pl.when(pl.program_id(0) == 0)
    def _():
        s_ref[...] = jnp.zeros_like(s_ref)
        c_ref[...] = jnp.zeros_like(c_ref)
    s_ref[0:1, :] += jnp.sum(r3, axis=0, keepdims=True)
    c_ref[...] += lax.dot_general(r3, r3, (((0,), (0,)), ((), ())),
                                  preferred_element_type=jnp.float32)


def _k_final(gx_ref, np_ref, fz_ref, w3_ref, c3_ref, w4_ref, c4_ref,
             a2_ref, c2_ref, o_ref):
    dp = _dp_tile(gx_ref, np_ref)
    r3 = jnp.maximum(
        lax.dot_general(dp, w3_ref[...], (((1,), (1,)), ((), ())),
                        preferred_element_type=jnp.float32) + c3_ref[...], 0.0)
    pe = jnp.maximum(
        lax.dot_general(r3, w4_ref[...], (((1,), (1,)), ((), ())),
                        preferred_element_type=jnp.float32) + c4_ref[...], 0.0)
    fj = jnp.maximum(fz_ref[...] * a2_ref[...] + c2_ref[...], 0.0)
    v = pe + fj
    o_ref[...] = jnp.max(v.reshape(RT // NS, NS, COUT), axis=1)


@functools.cache
def _build_sc_gather_np():
    mesh = plsc.VectorSubcoreMesh(core_axis_name="c", subcore_axis_name="s")

    @functools.partial(
        pl.kernel, mesh=mesh,
        out_type=jax.ShapeDtypeStruct((B * M, PAD), jnp.float32),
        scratch_types=[pltpu.VMEM((CH,), jnp.int32),
                       pltpu.VMEM((CH, PAD), jnp.float32),
                       pltpu.SemaphoreType.DMA])
    def gather_np(ptab, sidx, out, idx_v, rows_v, sem):
        wid = lax.axis_index("s") * NC + lax.axis_index("c")
        base = wid * (B * M // NW)
        pltpu.sync_copy(sidx.at[pl.ds(base, CH)], idx_v)
        pltpu.async_copy(ptab.at[idx_v], rows_v, sem).wait()
        pltpu.sync_copy(rows_v, out.at[pl.ds(base, CH)])

    return gather_np


@functools.cache
def _build_sc_gather_fj():
    mesh = plsc.VectorSubcoreMesh(core_axis_name="c", subcore_axis_name="s")

    @functools.partial(
        pl.kernel, mesh=mesh,
        out_type=[jax.ShapeDtypeStruct((NT2, COUT), jnp.float32),
                  jax.ShapeDtypeStruct((NT2, PAD), jnp.float32)],
        scratch_types=[pltpu.VMEM((CH,), jnp.int32),
                       pltpu.VMEM((CH, COUT), jnp.float32),
                       pltpu.VMEM((CH, PAD), jnp.float32),
                       pltpu.SemaphoreType.DMA,
                       pltpu.SemaphoreType.DMA])
    def gather_fj(ztab, ptab, gidx, outz, outp, idx_v, zrows_v, prows_v,
                  semz, semp):
        wid = lax.axis_index("s") * NC + lax.axis_index("c")
        rpw = NT2 // NW
        base = wid * rpw

        def body(i, carry):
            off = base + i * CH
            pltpu.sync_copy(gidx.at[pl.ds(off, CH)], idx_v)
            cz = pltpu.async_copy(ztab.at[idx_v], zrows_v, semz)
            cp = pltpu.async_copy(ptab.at[idx_v], prows_v, semp)
            cz.wait()
            cp.wait()
            pltpu.sync_copy(zrows_v, outz.at[pl.ds(off, CH)])
            pltpu.sync_copy(prows_v, outp.at[pl.ds(off, CH)])
            return carry

        lax.fori_loop(0, rpw // CH, body, 0)

    return gather_fj


def _sc_gather_np(ptab, sidx):
    return _build_sc_gather_np()(ptab, sidx)


def _sc_gather_fj(ztab, ptab, gidx):
    return _build_sc_gather_fj()(ztab, ptab, gidx)


def _fold_bn(s, q, n, g, b):
    mu = s / n
    var = q / n - mu * mu
    a = g / jnp.sqrt(var + EPS)
    return a, b - a * mu


def kernel(p, f, sample_idx, c1w1, g1, b1, c1w2, g2, b2,
           c2w1, g3, b3, c2w2, g4, b4):
    f32 = jnp.float32
    f2d = f.transpose(1, 0, 2).reshape(CIN, BN_)

    sq1 = pl.pallas_call(
        _k_stats1,
        grid=(BN_ // TN1,),
        in_specs=[pl.BlockSpec((CIN, CIN), lambda i: (0, 0)),
                  pl.BlockSpec((CIN, TN1), lambda i: (0, i))],
        out_specs=pl.BlockSpec((8, CIN), lambda i: (0, 0)),
        out_shape=jax.ShapeDtypeStruct((8, CIN), f32),
    )(c1w1, f2d)
    a1, c1v = _fold_bn(sq1[0], sq1[1], BN_, g1, b1)
    w1a = a1[:, None] * c1w1

    z2t, sq2 = pl.pallas_call(
        _k_mlp1,
        grid=(BN_ // TN2,),
        in_specs=[pl.BlockSpec((CIN, CIN), lambda i: (0, 0)),
                  pl.BlockSpec((CIN, 1), lambda i: (0, 0)),
                  pl.BlockSpec((COUT, CMID), lambda i: (0, 0)),
                  pl.BlockSpec((CIN, TN2), lambda i: (0, i))],
        out_specs=[pl.BlockSpec((TN2, COUT), lambda i: (i, 0)),
                   pl.BlockSpec((8, COUT), lambda i: (0, 0))],
        out_shape=[jax.ShapeDtypeStruct((BN_, COUT), f32),
                   jax.ShapeDtypeStruct((8, COUT), f32)],
    )(w1a, c1v[:, None], c1w2, f2d)
    a2, c2v = _fold_bn(sq2[0], sq2[1], BN_, g2, b2)

    p_pad = jnp.pad(p.reshape(BN_, 3), ((0, 0), (0, PAD - 3)))
    sidxg = (sample_idx.astype(jnp.int32)
             + (jnp.arange(B, dtype=jnp.int32) * N)[:, None]).reshape(B * M)
    npg = _sc_gather_np(p_pad, sidxg)
    new_p = npg[:, :3].reshape(B, M, 3)

    pt8 = jnp.pad(p.transpose(0, 2, 1), ((0, 0), (0, 5), (0, 0)))
    idx2d = pl.pallas_call(
        _k_ball,
        grid=(B, M // TM),
        in_specs=[pl.BlockSpec((1, 8, N), lambda b, t: (b, 0, 0)),
                  pl.BlockSpec((TM, PAD), lambda b, t: (b * (M // TM) + t, 0))],
        out_specs=pl.BlockSpec((TM, NS), lambda b, t: (b * (M // TM) + t, 0)),
        out_shape=jax.ShapeDtypeStruct((B * M, NS), jnp.int32),
    )(pt8, npg)
    gidx = idx2d.reshape(NT2)

    fz2, gxp = _sc_gather_fj(z2t, p_pad, gidx)

    row_specs = [pl.BlockSpec((RT, PAD), lambda i: (i, 0)),
                 pl.BlockSpec((RT // NS, PAD), lambda i: (i, 0))]
    sdp, cdp = pl.pallas_call(
        _k_dpstats,
        grid=(NT2 // RT,),
        in_specs=row_specs,
        out_specs=[pl.BlockSpec((8, PAD), lambda i: (0, 0)),
                   pl.BlockSpec((PAD, PAD), lambda i: (0, 0))],
        out_shape=[jax.ShapeDtypeStruct((8, PAD), f32),
                   jax.ShapeDtypeStruct((PAD, PAD), f32)],
    )(gxp, npg)
    w3p = jnp.zeros((CMID, PAD), f32).at[:, :3].set(c2w1)
    mu3 = w3p @ (sdp[0] / NT2)
    e3 = jnp.sum((w3p @ (cdp / NT2)) * w3p, axis=1)
    a3 = g3 / jnp.sqrt(e3 - mu3 * mu3 + EPS)
    c3v = b3 - a3 * mu3
    w3a = a3[:, None] * w3p

    s3, c3m = pl.pallas_call(
        _k_r3stats,
        grid=(NT2 // RT,),
        in_specs=row_specs + [pl.BlockSpec((CMID, PAD), lambda i: (0, 0)),
                              pl.BlockSpec((1, CMID), lambda i: (0, 0))],
        out_specs=[pl.BlockSpec((8, CMID), lambda i: (0, 0)),
                   pl.BlockSpec((CMID, CMID), lambda i: (0, 0))],
        out_shape=[jax.ShapeDtypeStruct((8, CMID), f32),
                   jax.ShapeDtypeStruct((CMID, CMID), f32)],
    )(gxp, npg, w3a, c3v[None, :])
    mu4 = c2w2 @ (s3[0] / NT2)
    e4 = jnp.sum((c2w2 @ (c3m / NT2)) * c2w2, axis=1)
    a4 = g4 / jnp.sqrt(e4 - mu4 * mu4 + EPS)
    c4v = b4 - a4 * mu4
    w4a = a4[:, None] * c2w2

    out2d = pl.pallas_call(
        _k_final,
        grid=(NT2 // RT,),
        in_specs=row_specs + [
            pl.BlockSpec((RT, COUT), lambda i: (i, 0)),
            pl.BlockSpec((CMID, PAD), lambda i: (0, 0)),
            pl.BlockSpec((1, CMID), lambda i: (0, 0)),
            pl.BlockSpec((COUT, CMID), lambda i: (0, 0)),
            pl.BlockSpec((1, COUT), lambda i: (0, 0)),
            pl.BlockSpec((1, COUT), lambda i: (0, 0)),
            pl.BlockSpec((1, COUT), lambda i: (0, 0))],
        out_specs=pl.BlockSpec((RT // NS, COUT), lambda i: (i, 0)),
        out_shape=jax.ShapeDtypeStruct((B * M, COUT), f32),
    )(gxp, npg, fz2, w3a, c3v[None, :], w4a, c4v[None, :],
      a2[None, :], c2v[None, :])

    out = out2d.reshape(B, M, COUT).transpose(0, 2, 1)
    return (new_p, out)


# speedup vs baseline: 9.3660x; 1.0514x over previous
import functools

import jax
import jax.numpy as jnp
import numpy as np
from jax import lax
from jax.experimental import pallas as pl
from jax.experimental.pallas import tpu as pltpu
from jax.experimental.pallas import tpu_sc as plsc

B = 2
N = 8192
M = N // 4
NS = 32
CIN = 128
CMID = 128
COUT = 256
RADIUS = 0.1
EPS = 1e-5
BN_ = B * N
NT2 = B * M * NS
PAD = 128

R2 = np.float32(RADIUS * RADIUS)

TN1 = 2048
TN2 = 1024
TM = 128
RT = 1024

NC = 2
NSUB = 16
NW = NC * NSUB
CH = 128


def _k_stats1(w1_ref, f_ref, sq_ref):
    z = jnp.dot(w1_ref[...], f_ref[...], preferred_element_type=jnp.float32)
    @pl.when(pl.program_id(0) == 0)
    def _():
        sq_ref[...] = jnp.zeros_like(sq_ref)
    sq_ref[0:1, :] += jnp.sum(z, axis=1, keepdims=True).T
    sq_ref[1:2, :] += jnp.sum(z * z, axis=1, keepdims=True).T


def _k_mlp1(w1a_ref, c1_ref, w2_ref, f_ref, z2t_ref, sq_ref):
    z1 = jnp.dot(w1a_ref[...], f_ref[...], preferred_element_type=jnp.float32)
    r = jnp.maximum(z1 + c1_ref[...], 0.0)
    z2t = lax.dot_general(r, w2_ref[...], (((0,), (1,)), ((), ())),
                          preferred_element_type=jnp.float32)
    z2t_ref[...] = z2t
    @pl.when(pl.program_id(0) == 0)
    def _():
        sq_ref[...] = jnp.zeros_like(sq_ref)
    sq_ref[0:1, :] += jnp.sum(z2t, axis=0, keepdims=True)
    sq_ref[1:2, :] += jnp.sum(z2t * z2t, axis=0, keepdims=True)


def _k_ball(pt_ref, c_ref, idx_ref):
    b = pl.program_id(0)
    pt = pt_ref[0]
    d = None
    for c in range(3):
        diff = c_ref[:, c:c + 1] - pt[c:c + 1, :]
        sq = diff * diff
        d = sq if d is None else d + sq
    hits = jnp.where(d <= R2, 1.0, 0.0)
    cs = hits
    sh = 1
    while sh < N:
        cs = cs + jnp.concatenate(
            [jnp.zeros((TM, sh), jnp.float32), cs[:, :N - sh]], axis=1)
        sh *= 2
    ones_col = jnp.ones((N, 1), jnp.float32)
    cols = []
    for k in range(NS):
        cmpf = jnp.where(cs <= jnp.float32(k), 1.0, 0.0)
        cols.append(jnp.dot(cmpf, ones_col,
                            preferred_element_type=jnp.float32))
    idx = jnp.concatenate(cols, axis=1).astype(jnp.int32)
    first = idx[:, 0:1]
    idx = jnp.where(idx == N, first, idx)
    idx_ref[...] = idx + b * N


def _dp_tile(gx_ref, np_ref):
    gx = gx_ref[...]
    npv = np_ref[...]
    dp = (gx.reshape(RT // NS, NS, PAD) - npv[:, None, :]) / jnp.float32(RADIUS)
    return dp.reshape(RT, PAD)


def _k_dpstats(gx_ref, np_ref, s_ref, c_ref):
    dp = _dp_tile(gx_ref, np_ref)
    @pl.when(pl.program_id(0) == 0)
    def _():
        s_ref[...] = jnp.zeros_like(s_ref)
        c_ref[...] = jnp.zeros_like(c_ref)
    s_ref[0:1, :] += jnp.sum(dp, axis=0, keepdims=True)
    c_ref[...] += lax.dot_general(dp, dp, (((0,), (0,)), ((), ())),
                                  preferred_element_type=jnp.float32)


def _k_r3stats(gx_ref, np_ref, w3_ref, c3_ref, s_ref, c_ref):
    dp = _dp_tile(gx_ref, np_ref)
    r3 = jnp.maximum(
        lax.dot_general(dp, w3_ref[...], (((1,), (1,)), ((), ())),
                        preferred_element_type=jnp.float32) + c3_ref[...], 0.0)
    @pl.when(pl.program_id(0) == 0)
    def _():
        s_ref[...] = jnp.zeros_like(s_ref)
        c_ref[...] = jnp.zeros_like(c_ref)
    s_ref[0:1, :] += jnp.sum(r3, axis=0, keepdims=True)
    c_ref[...] += lax.dot_general(r3, r3, (((0,), (0,)), ((), ())),
                                  preferred_element_type=jnp.float32)


def _k_final(gx_ref, np_ref, fz_ref, w3_ref, c3_ref, w4_ref, c4_ref,
             a2_ref, c2_ref, o_ref):
    dp = _dp_tile(gx_ref, np_ref)
    r3 = jnp.maximum(
        lax.dot_general(dp, w3_ref[...], (((1,), (1,)), ((), ())),
                        preferred_element_type=jnp.float32) + c3_ref[...], 0.0)
    pe = jnp.maximum(
        lax.dot_general(r3, w4_ref[...], (((1,), (1,)), ((), ())),
                        preferred_element_type=jnp.float32) + c4_ref[...], 0.0)
    fj = jnp.maximum(fz_ref[...] * a2_ref[...] + c2_ref[...], 0.0)
    v = pe + fj
    o_ref[...] = jnp.max(v.reshape(RT // NS, NS, COUT), axis=1)


@functools.cache
def _build_sc_gather_np():
    mesh = plsc.VectorSubcoreMesh(core_axis_name="c", subcore_axis_name="s")

    @functools.partial(
        pl.kernel, mesh=mesh,
        out_type=jax.ShapeDtypeStruct((B * M, PAD), jnp.float32),
        scratch_types=[pltpu.VMEM((CH,), jnp.int32),
                       pltpu.VMEM((CH, PAD), jnp.float32),
                       pltpu.SemaphoreType.DMA])
    def gather_np(ptab, sidx, out, idx_v, rows_v, sem):
        wid = lax.axis_index("s") * NC + lax.axis_index("c")
        base = wid * (B * M // NW)
        pltpu.sync_copy(sidx.at[pl.ds(base, CH)], idx_v)
        pltpu.async_copy(ptab.at[idx_v], rows_v, sem).wait()
        pltpu.sync_copy(rows_v, out.at[pl.ds(base, CH)])

    return gather_np


@functools.cache
def _build_sc_gather_fj():
    mesh = plsc.VectorSubcoreMesh(core_axis_name="c", subcore_axis_name="s")

    @functools.partial(
        pl.kernel, mesh=mesh,
        out_type=[jax.ShapeDtypeStruct((NT2, COUT), jnp.float32),
                  jax.ShapeDtypeStruct((NT2, PAD), jnp.float32)],
        scratch_types=[pltpu.VMEM((CH,), jnp.int32),
                       pltpu.VMEM((CH, COUT), jnp.float32),
                       pltpu.VMEM((CH, PAD), jnp.float32),
                       pltpu.SemaphoreType.DMA,
                       pltpu.SemaphoreType.DMA])
    def gather_fj(ztab, ptab, gidx, outz, outp, idx_v, zrows_v, prows_v,
                  semz, semp):
        wid = lax.axis_index("s") * NC + lax.axis_index("c")
        rpw = NT2 // NW
        base = wid * rpw

        def body(i, carry):
            off = base + i * CH
            pltpu.sync_copy(gidx.at[pl.ds(off, CH)], idx_v)
            cz = pltpu.async_copy(ztab.at[idx_v], zrows_v, semz)
            cp = pltpu.async_copy(ptab.at[idx_v], prows_v, semp)
            cz.wait()
            cp.wait()
            pltpu.sync_copy(zrows_v, outz.at[pl.ds(off, CH)])
            pltpu.sync_copy(prows_v, outp.at[pl.ds(off, CH)])
            return carry

        lax.fori_loop(0, rpw // CH, body, 0)

    return gather_fj


def _sc_gather_np(ptab, sidx):
    return _build_sc_gather_np()(ptab, sidx)


def _sc_gather_fj(ztab, ptab, gidx):
    return _build_sc_gather_fj()(ztab, ptab, gidx)


def _fold_bn(s, q, n, g, b):
    mu = s / n
    var = q / n - mu * mu
    a = g / jnp.sqrt(var + EPS)
    return a, b - a * mu


def kernel(p, f, sample_idx, c1w1, g1, b1, c1w2, g2, b2,
           c2w1, g3, b3, c2w2, g4, b4):
    f32 = jnp.float32
    f2d = f.transpose(1, 0, 2).reshape(CIN, BN_)

    sq1 = pl.pallas_call(
        _k_stats1,
        grid=(BN_ // TN1,),
        in_specs=[pl.BlockSpec((CIN, CIN), lambda i: (0, 0)),
                  pl.BlockSpec((CIN, TN1), lambda i: (0, i))],
        out_specs=pl.BlockSpec((8, CIN), lambda i: (0, 0)),
        out_shape=jax.ShapeDtypeStruct((8, CIN), f32),
    )(c1w1, f2d)
    a1, c1v = _fold_bn(sq1[0], sq1[1], BN_, g1, b1)
    w1a = a1[:, None] * c1w1

    z2t, sq2 = pl.pallas_call(
        _k_mlp1,
        grid=(BN_ // TN2,),
        in_specs=[pl.BlockSpec((CIN, CIN), lambda i: (0, 0)),
                  pl.BlockSpec((CIN, 1), lambda i: (0, 0)),
                  pl.BlockSpec((COUT, CMID), lambda i: (0, 0)),
                  pl.BlockSpec((CIN, TN2), lambda i: (0, i))],
        out_specs=[pl.BlockSpec((TN2, COUT), lambda i: (i, 0)),
                   pl.BlockSpec((8, COUT), lambda i: (0, 0))],
        out_shape=[jax.ShapeDtypeStruct((BN_, COUT), f32),
                   jax.ShapeDtypeStruct((8, COUT), f32)],
    )(w1a, c1v[:, None], c1w2, f2d)
    a2, c2v = _fold_bn(sq2[0], sq2[1], BN_, g2, b2)

    p_pad = jnp.pad(p.reshape(BN_, 3), ((0, 0), (0, PAD - 3)))
    sidxg = (sample_idx.astype(jnp.int32)
             + (jnp.arange(B, dtype=jnp.int32) * N)[:, None]).reshape(B * M)
    npg = _sc_gather_np(p_pad, sidxg)
    new_p = npg[:, :3].reshape(B, M, 3)

    pt8 = jnp.pad(p.transpose(0, 2, 1), ((0, 0), (0, 5), (0, 0)))
    idx2d = pl.pallas_call(
        _k_ball,
        grid=(B, M // TM),
        in_specs=[pl.BlockSpec((1, 8, N), lambda b, t: (b, 0, 0)),
                  pl.BlockSpec((TM, PAD), lambda b, t: (b * (M // TM) + t, 0))],
        out_specs=pl.BlockSpec((TM, NS), lambda b, t: (b * (M // TM) + t, 0)),
        out_shape=jax.ShapeDtypeStruct((B * M, NS), jnp.int32),
    )(pt8, npg)
    gidx = idx2d.reshape(NT2)

    fz2, gxp = _sc_gather_fj(z2t, p_pad, gidx)

    row_specs = [pl.BlockSpec((RT, PAD), lambda i: (i, 0)),
                 pl.BlockSpec((RT // NS, PAD), lambda i: (i, 0))]
    sdp, cdp = pl.pallas_call(
        _k_dpstats,
        grid=(NT2 // RT,),
        in_specs=row_specs,
        out_specs=[pl.BlockSpec((8, PAD), lambda i: (0, 0)),
                   pl.BlockSpec((PAD, PAD), lambda i: (0, 0))],
        out_shape=[jax.ShapeDtypeStruct((8, PAD), f32),
                   jax.ShapeDtypeStruct((PAD, PAD), f32)],
    )(gxp, npg)
    w3p = jnp.zeros((CMID, PAD), f32).at[:, :3].set(c2w1)
    mu3 = w3p @ (sdp[0] / NT2)
    e3 = jnp.sum((w3p @ (cdp / NT2)) * w3p, axis=1)
    a3 = g3 / jnp.sqrt(e3 - mu3 * mu3 + EPS)
    c3v = b3 - a3 * mu3
    w3a = a3[:, None] * w3p

    s3, c3m = pl.pallas_call(
        _k_r3stats,
        grid=(NT2 // RT,),
        in_specs=row_specs + [pl.BlockSpec((CMID, PAD), lambda i: (0, 0)),
                              pl.BlockSpec((1, CMID), lambda i: (0, 0))],
        out_specs=[pl.BlockSpec((8, CMID), lambda i: (0, 0)),
                   pl.BlockSpec((CMID, CMID), lambda i: (0, 0))],
        out_shape=[jax.ShapeDtypeStruct((8, CMID), f32),
                   jax.ShapeDtypeStruct((CMID, CMID), f32)],
    )(gxp, npg, w3a, c3v[None, :])
    mu4 = c2w2 @ (s3[0] / NT2)
    e4 = jnp.sum((c2w2 @ (c3m / NT2)) * c2w2, axis=1)
    a4 = g4 / jnp.sqrt(e4 - mu4 * mu4 + EPS)
    c4v = b4 - a4 * mu4
    w4a = a4[:, None] * c2w2

    out2d = pl.pallas_call(
        _k_final,
        grid=(NT2 // RT,),
        in_specs=row_specs + [
            pl.BlockSpec((RT, COUT), lambda i: (i, 0)),
            pl.BlockSpec((CMID, PAD), lambda i: (0, 0)),
            pl.BlockSpec((1, CMID), lambda i: (0, 0)),
            pl.BlockSpec((COUT, CMID), lambda i: (0, 0)),
            pl.BlockSpec((1, COUT), lambda i: (0, 0)),
            pl.BlockSpec((1, COUT), lambda i: (0, 0)),
            pl.BlockSpec((1, COUT), lambda i: (0, 0))],
        out_specs=pl.BlockSpec((RT // NS, COUT), lambda i: (i, 0)),
        out_shape=jax.ShapeDtypeStruct((B * M, COUT), f32),
    )(gxp, npg, fz2, w3a, c3v[None, :], w4a, c4v[None, :],
      a2[None, :], c2v[None, :])

    out = out2d.reshape(B, M, COUT).transpose(0, 2, 1)
    return (new_p, out)
